# baseline (speedup 1.0000x reference)
import functools

import jax
import jax.numpy as jnp
from jax import lax
from jax.experimental import pallas as pl
from jax.experimental.pallas import tpu as pltpu
from jax.experimental.pallas import tpu_sc as plsc

B = 16384
H = 50
D = 32
NW = 32
CB = B // NW


def _sqrt16(x):
    xs = jnp.maximum(x, 1e-12)
    i = lax.bitcast_convert_type(xs, jnp.int32)
    y = lax.bitcast_convert_type(jnp.int32(0x5F3759DF) - (i >> 1), jnp.float32)
    y = y * (1.5 - 0.5 * xs * y * y)
    y = y * (1.5 - 0.5 * xs * y * y)
    return xs * y


def _sc_body(hist_hbm, dist_hbm, hj_hbm, gi_hbm, out_hbm,
             hist_v, dq_v, hj_v, gA, gB, w_v, out_v, semA, semB):
    wid = lax.axis_index("c") * 16 + lax.axis_index("s")
    base = wid * CB

    pltpu.sync_copy(hist_hbm.at[pl.ds(base, CB)], hist_v)
    pltpu.sync_copy(dist_hbm.at[pl.ds(base, CB)], dq_v)
    pltpu.sync_copy(hj_hbm.at[pl.ds(base, CB)], hj_v)

    def start(r, buf, sem):
        pltpu.async_copy(gi_hbm.at[hist_v.at[r]], buf, sem)

    def wait(r, buf, sem):
        pltpu.make_async_copy(gi_hbm.at[hist_v.at[r]], buf, sem).wait()

    start(0, gA, semA)
    start(1, gB, semB)

    _offs = (0, 16, 32, 34)

    def compute_row(buf, r):
        hj0 = hj_v[r, pl.ds(0, 16)]
        hj1 = hj_v[r, pl.ds(16, 16)]
        f = [_sqrt16(dq_v[r, pl.ds(o, 16)]) for o in _offs]
        for d in range(D):
            hv = hj0 if d < 16 else hj1
            hjd = jnp.broadcast_to(hv[d % 16], (16,))
            for t in range(4):
                w_v[pl.ds(50 * d + _offs[t], 16)] = hjd * f[t]

        def e_step(e, accy):
            k0 = e * 32
            g0 = buf[e, pl.ds(0, 16)]
            w0 = w_v[pl.ds(k0, 16)]
            g1 = buf[e, pl.ds(16, 16)]
            w1 = w_v[pl.ds(k0 + 16, 16)]
            return accy + g0 * w0 + g1 * w1

        accy = lax.fori_loop(0, H, e_step, jnp.zeros((16,), jnp.float32))
        out_v[r, pl.ds(0, 16)] = accy * (1.0 / H)

    def body(j, c):
        r = 2 * j
        wait(r, gA, semA)
        compute_row(gA, r)

        @pl.when(j < CB // 2 - 1)
        def _():
            start(r + 2, gA, semA)

        wait(r + 1, gB, semB)
        compute_row(gB, r + 1)

        @pl.when(j < CB // 2 - 1)
        def _():
            start(r + 3, gB, semB)
        return c

    lax.fori_loop(0, CB // 2, body, 0)
    pltpu.sync_copy(out_v, out_hbm.at[pl.ds(base, CB)])


def _fin_body(part_ref, up_ref, pp_ref, cuj_ref, out_s_ref, out_w_ref):
    yij = jnp.sum(part_ref[...], axis=1, keepdims=True)
    tz = jnp.sum(up_ref[...] * pp_ref[...], axis=1, keepdims=True)
    suj = tz + yij
    out_s_ref[...] = 1.0 / (1.0 + jnp.exp(-suj))
    out_w_ref[...] = 1.0 + jnp.log(1.0 + cuj_ref[...] * (10.0 ** 10))


def kernel(user_id, targets, history, check_in_num, distances,
           UserPreference, PoiPreference, GeoInfluence, GeoSusceptibility):
    hist = history.astype(jnp.int32)
    hj = jnp.take(GeoSusceptibility, targets, axis=0)
    up = jnp.take(UserPreference, user_id, axis=0)
    pp = jnp.take(PoiPreference, targets, axis=0)

    mesh = plsc.VectorSubcoreMesh(core_axis_name="c", subcore_axis_name="s")
    sc = pl.kernel(
        _sc_body,
        mesh=mesh,
        compiler_params=pltpu.CompilerParams(use_tc_tiling_on_sc=False),
        out_type=jax.ShapeDtypeStruct((B, 16), jnp.float32),
        scratch_types=[
            pltpu.VMEM((CB, H), jnp.int32),
            pltpu.VMEM((CB, H), jnp.float32),
            pltpu.VMEM((CB, D), jnp.float32),
            pltpu.VMEM((H, D), jnp.float32),
            pltpu.VMEM((H, D), jnp.float32),
            pltpu.VMEM((1600,), jnp.float32),
            pltpu.VMEM((CB, 16), jnp.float32),
            pltpu.SemaphoreType.DMA,
            pltpu.SemaphoreType.DMA,
        ],
    )
    part = sc(hist, distances, hj, GeoInfluence)

    out_s, wuj = pl.pallas_call(
        _fin_body,
        grid=(8,),
        in_specs=[
            pl.BlockSpec((B // 8, 16), lambda i: (i, 0)),
            pl.BlockSpec((B // 8, D), lambda i: (i, 0)),
            pl.BlockSpec((B // 8, D), lambda i: (i, 0)),
            pl.BlockSpec((B // 8, 1), lambda i: (i, 0)),
        ],
        out_specs=[
            pl.BlockSpec((B // 8, 1), lambda i: (i, 0)),
            pl.BlockSpec((B // 8, 1), lambda i: (i, 0)),
        ],
        out_shape=[
            jax.ShapeDtypeStruct((B, 1), jnp.float32),
            jax.ShapeDtypeStruct((B, 1), jnp.float32),
        ],
    )(part, up, pp, check_in_num)

    return out_s, wuj


# speedup vs baseline: 1.1024x; 1.1024x over previous
import functools

import jax
import jax.numpy as jnp
from jax import lax
from jax.experimental import pallas as pl
from jax.experimental.pallas import tpu as pltpu
from jax.experimental.pallas import tpu_sc as plsc

B = 16384
H = 50
D = 32
DP = 64
NW = 32
CB = B // NW
NPAIR = CB // 2


def _sqrt16(x):
    xs = jnp.maximum(x, 1e-12)
    i = lax.bitcast_convert_type(xs, jnp.int32)
    y = lax.bitcast_convert_type(jnp.int32(0x5F3759DF) - (i >> 1), jnp.float32)
    y = y * (1.5 - 0.5 * xs * y * y)
    y = y * (1.5 - 0.5 * xs * y * y)
    return xs * y


def _sc_body(hist_hbm, dist_hbm, hj_hbm, gi_hbm, out_hbm,
             hist_v, dpf_v, hj_v, gA, gB, w_v, out_v, semA, semB):
    wid = lax.axis_index("c") * 16 + lax.axis_index("s")
    base = wid * CB

    pltpu.sync_copy(hist_hbm.at[pl.ds(wid * NPAIR, NPAIR)], hist_v)
    pltpu.sync_copy(dist_hbm.at[pl.ds(base * DP, CB * DP)], dpf_v)
    pltpu.sync_copy(hj_hbm.at[pl.ds(base, CB)], hj_v)

    def _sqrt_step(i, c):
        sl = pl.ds(i * 16, 16)
        dpf_v[sl] = _sqrt16(dpf_v[sl])
        return c
    lax.fori_loop(0, CB * DP // 16, _sqrt_step, 0, unroll=4)

    def start(p, buf, sem):
        pltpu.async_copy(gi_hbm.at[hist_v.at[p]], buf, sem)

    def wait(p, buf, sem):
        pltpu.make_async_copy(gi_hbm.at[hist_v.at[p]], buf, sem).wait()

    start(0, gA, semA)
    start(1, gB, semB)

    def compute_row(buf, r, off):
        hj0 = hj_v[r, pl.ds(0, 16)]
        hj1 = hj_v[r, pl.ds(16, 16)]
        rb = r * DP
        f = [dpf_v[pl.ds(rb + 16 * t, 16)] for t in range(4)]
        for d in range(D):
            hv = hj0 if d < 16 else hj1
            hjd = jnp.broadcast_to(hv[d % 16], (16,))
            for t in range(4):
                w_v[pl.ds(50 * d + 16 * t, 16)] = hjd * f[t]

        def e_step(e, accy):
            er = off + e
            k0 = e * 32
            g0 = buf[er, pl.ds(0, 16)]
            w0 = w_v[pl.ds(k0, 16)]
            g1 = buf[er, pl.ds(16, 16)]
            w1 = w_v[pl.ds(k0 + 16, 16)]
            return accy + g0 * w0 + g1 * w1

        accy = lax.fori_loop(0, H, e_step, jnp.zeros((16,), jnp.float32),
                             unroll=5)
        out_v[r, pl.ds(0, 16)] = accy * (1.0 / H)

    def body(j, c):
        p = 2 * j
        wait(p, gA, semA)
        compute_row(gA, 2 * p, 0)
        compute_row(gA, 2 * p + 1, H)

        @pl.when(j < NPAIR // 2 - 1)
        def _():
            start(p + 2, gA, semA)

        wait(p + 1, gB, semB)
        compute_row(gB, 2 * p + 2, 0)
        compute_row(gB, 2 * p + 3, H)

        @pl.when(j < NPAIR // 2 - 1)
        def _():
            start(p + 3, gB, semB)
        return c

    lax.fori_loop(0, NPAIR // 2, body, 0)
    pltpu.sync_copy(out_v, out_hbm.at[pl.ds(base, CB)])


def _fin_body(part_ref, up_ref, pp_ref, cuj_ref, out_s_ref, out_w_ref):
    yij = jnp.sum(part_ref[...], axis=1, keepdims=True)
    tz = jnp.sum(up_ref[...] * pp_ref[...], axis=1, keepdims=True)
    suj = tz + yij
    out_s_ref[...] = 1.0 / (1.0 + jnp.exp(-suj))
    out_w_ref[...] = 1.0 + jnp.log(1.0 + cuj_ref[...] * (10.0 ** 10))


def kernel(user_id, targets, history, check_in_num, distances,
           UserPreference, PoiPreference, GeoInfluence, GeoSusceptibility):
    hist2 = history.astype(jnp.int32).reshape(B // 2, 2 * H)
    dist_flat = jnp.pad(distances, ((0, 0), (0, DP - H))).reshape(B * DP)
    hj = jnp.take(GeoSusceptibility, targets, axis=0)
    up = jnp.take(UserPreference, user_id, axis=0)
    pp = jnp.take(PoiPreference, targets, axis=0)

    mesh = plsc.VectorSubcoreMesh(core_axis_name="c", subcore_axis_name="s")
    sc = pl.kernel(
        _sc_body,
        mesh=mesh,
        compiler_params=pltpu.CompilerParams(use_tc_tiling_on_sc=False),
        out_type=jax.ShapeDtypeStruct((B, 16), jnp.float32),
        scratch_types=[
            pltpu.VMEM((NPAIR, 2 * H), jnp.int32),
            pltpu.VMEM((CB * DP,), jnp.float32),
            pltpu.VMEM((CB, D), jnp.float32),
            pltpu.VMEM((2 * H, D), jnp.float32),
            pltpu.VMEM((2 * H, D), jnp.float32),
            pltpu.VMEM((1664,), jnp.float32),
            pltpu.VMEM((CB, 16), jnp.float32),
            pltpu.SemaphoreType.DMA,
            pltpu.SemaphoreType.DMA,
        ],
    )
    part = sc(hist2, dist_flat, hj, GeoInfluence)

    out_s, wuj = pl.pallas_call(
        _fin_body,
        grid=(8,),
        in_specs=[
            pl.BlockSpec((B // 8, 16), lambda i: (i, 0)),
            pl.BlockSpec((B // 8, D), lambda i: (i, 0)),
            pl.BlockSpec((B // 8, D), lambda i: (i, 0)),
            pl.BlockSpec((B // 8, 1), lambda i: (i, 0)),
        ],
        out_specs=[
            pl.BlockSpec((B // 8, 1), lambda i: (i, 0)),
            pl.BlockSpec((B // 8, 1), lambda i: (i, 0)),
        ],
        out_shape=[
            jax.ShapeDtypeStruct((B, 1), jnp.float32),
            jax.ShapeDtypeStruct((B, 1), jnp.float32),
        ],
    )(part, up, pp, check_in_num)

    return out_s, wuj
